# baseline (speedup 1.0000x reference)
import functools

import jax
import jax.numpy as jnp
from jax.experimental import pallas as pl
from jax.experimental.pallas import tpu as pltpu


def _gap_sum_kernel(x_ref, o_ref, *, hw, inv_hw):
    acc = x_ref[0]
    for k in range(1, hw):
        acc = acc + x_ref[k]
    o_ref[...] = (acc * inv_hw).astype(o_ref.dtype)


def kernel(x):
    n, c, h, w = x.shape
    hw = h * w

    xt = jnp.transpose(x, (2, 3, 0, 1)).reshape(hw, n, c)

    tc = min(128, c)
    grid_c = c // tc

    out = pl.pallas_call(
        functools.partial(_gap_sum_kernel, hw=hw, inv_hw=1.0 / hw),
        out_shape=jax.ShapeDtypeStruct((n, c), x.dtype),
        grid=(grid_c,),
        in_specs=[pl.BlockSpec((hw, n, tc), lambda j: (0, 0, j))],
        out_specs=pl.BlockSpec((n, tc), lambda j: (0, j)),
        compiler_params=pltpu.CompilerParams(
            dimension_semantics=("parallel",),
        ),
    )(xt)

    return out


# speedup vs baseline: 1.1142x; 1.1142x over previous
import functools

import jax
import jax.numpy as jnp
from jax.experimental import pallas as pl
from jax.experimental.pallas import tpu as pltpu


def _gap_sum_kernel(x_ref, o_ref, *, hw, inv_hw):
    acc = x_ref[0]
    for k in range(1, hw):
        acc = acc + x_ref[k]
    o_ref[...] = (acc * inv_hw).astype(o_ref.dtype)


def kernel(x):
    n, c, h, w = x.shape
    hw = h * w

    xt = jnp.transpose(x, (2, 3, 0, 1)).reshape(hw, n, c)

    tn = min(64, n)
    tc = min(512, c)
    grid_n = n // tn
    grid_c = c // tc

    out = pl.pallas_call(
        functools.partial(_gap_sum_kernel, hw=hw, inv_hw=1.0 / hw),
        out_shape=jax.ShapeDtypeStruct((n, c), x.dtype),
        grid=(grid_n, grid_c),
        in_specs=[pl.BlockSpec((hw, tn, tc), lambda i, j: (0, i, j))],
        out_specs=pl.BlockSpec((tn, tc), lambda i, j: (i, j)),
        compiler_params=pltpu.CompilerParams(
            dimension_semantics=("parallel", "parallel"),
        ),
    )(xt)

    return out


# speedup vs baseline: 1.1571x; 1.0385x over previous
import functools

import jax
import jax.numpy as jnp
from jax.experimental import pallas as pl
from jax.experimental.pallas import tpu as pltpu


def _gap_sum_kernel(x_ref, o_ref, *, hw, inv_hw):
    acc = x_ref[0]
    for k in range(1, hw):
        acc = acc + x_ref[k]
    o_ref[...] = (acc * inv_hw).astype(o_ref.dtype)


def kernel(x):
    n, c, h, w = x.shape
    hw = h * w

    xt = jnp.transpose(x, (2, 3, 0, 1)).reshape(hw, n, c)

    tc = min(256, c)
    grid_c = c // tc

    out = pl.pallas_call(
        functools.partial(_gap_sum_kernel, hw=hw, inv_hw=1.0 / hw),
        out_shape=jax.ShapeDtypeStruct((n, c), x.dtype),
        grid=(grid_c,),
        in_specs=[pl.BlockSpec((hw, n, tc), lambda j: (0, 0, j))],
        out_specs=pl.BlockSpec((n, tc), lambda j: (0, j)),
        compiler_params=pltpu.CompilerParams(
            dimension_semantics=("parallel",),
        ),
    )(xt)

    return out
